# baseline (speedup 1.0000x reference)
import functools

import jax
import jax.numpy as jnp
from jax import lax
from jax.experimental import pallas as pl
from jax.experimental.pallas import tpu as pltpu
from jax.experimental.pallas import tpu_sc as plsc

_N = 10000
_E = 320000
_MUL = 32
_EDGE_DIM = 12

_NC = 2
_NS = 16
_NW = _NC * _NS
_EPW = _E // _NW
_B = 40
_NB = _EPW // _B
_RPT = 624
_RTAIL = _N - _NS * _RPT

_INV_SQRT_MUL = 1.0 / (32.0 ** 0.5)
_INV_SQRT_EDGE = 1.0 / (12.0 ** 0.5)
_SQRT3 = 3.0 ** 0.5


def _node_linear_body(nf_ref, w_ref, x_ref):
    x_ref[...] = jnp.dot(nf_ref[...], w_ref[...],
                         preferred_element_type=jnp.float32) * _INV_SQRT_MUL


def _node_linear(nf, w_nl0):
    return pl.pallas_call(
        _node_linear_body,
        out_shape=jax.ShapeDtypeStruct((_N, _MUL), jnp.float32),
    )(nf, w_nl0)


_BE = 2560


def _edge_prep_body(embt_ref, vect_ref, wfc4_ref, ew_ref):
    vt = vect_ref[...]
    n2 = jnp.sum(vt * vt, axis=0, keepdims=True)
    s = _SQRT3 / (jnp.sqrt(n2) + 1e-12)
    y4 = jnp.concatenate([jnp.ones_like(s), vt * s], axis=0)
    dn = (((0,), (0,)), ((), ()))
    w128 = lax.dot_general(embt_ref[...], wfc4_ref[...], dn,
                           preferred_element_type=jnp.float32)
    row = lax.broadcasted_iota(jnp.int32, (4, 4 * _MUL), 0)
    col = lax.broadcasted_iota(jnp.int32, (4, 4 * _MUL), 1)
    sel4 = (col // _MUL == row).astype(jnp.float32)
    ybc = lax.dot_general(y4, sel4, dn,
                          preferred_element_type=jnp.float32)
    ew_ref[...] = (w128 * ybc).astype(jnp.bfloat16)


def _edge_prep(embt, vect, wfc4):
    grid = _E // _BE
    return pl.pallas_call(
        _edge_prep_body,
        grid=(grid,),
        in_specs=[
            pl.BlockSpec((_EDGE_DIM, _BE), lambda i: (0, i)),
            pl.BlockSpec((3, _BE), lambda i: (0, i)),
            pl.BlockSpec((_EDGE_DIM, 4 * _MUL), lambda i: (0, 0)),
        ],
        out_specs=pl.BlockSpec((_BE, 4 * _MUL), lambda i: (i, 0)),
        out_shape=jax.ShapeDtypeStruct((_E, 4 * _MUL), jnp.bfloat16),
    )(embt, vect, wfc4)


def _sc_body(x_hbm, ew_hbm, src_hbm, dst_hbm, z_hbm, out_hbm,
             srci, dsti, xd, ewv, pay, acc,
             sem_in0, sem_in1, sem_sc0, sem_sc1):
    cid = lax.axis_index("c")
    sid = lax.axis_index("s")
    wid = cid * _NS + sid
    row0 = wid * _NB

    sem_in = [sem_in0, sem_in1]
    sem_sc = [sem_sc0, sem_sc1]

    pltpu.sync_copy(src_hbm.at[pl.ds(row0, _NB)], srci)
    pltpu.sync_copy(dst_hbm.at[pl.ds(row0, _NB)], dsti)

    pltpu.sync_copy(z_hbm, acc.at[pl.ds(sid * _RPT, _RPT)])

    @pl.when(sid == 0)
    def _zero_tail():
        pltpu.sync_copy(z_hbm.at[pl.ds(0, _RTAIL)],
                        acc.at[pl.ds(_NS * _RPT, _RTAIL)])

    plsc.subcore_barrier()

    def start_in(i, s):
        pltpu.async_copy(x_hbm.at[dsti.at[i]], xd.at[s], sem_in[s])
        pltpu.async_copy(ew_hbm.at[pl.ds((row0 + i) * _B, _B)], ewv.at[s],
                         sem_in[s])

    def wait_in(i, s):
        pltpu.make_async_copy(x_hbm.at[dsti.at[i]], xd.at[s],
                              sem_in[s]).wait()
        pltpu.make_async_copy(ew_hbm.at[pl.ds((row0 + i) * _B, _B)],
                              ewv.at[s], sem_in[s]).wait()

    def start_scatter(i, s):
        pltpu.async_copy(pay.at[s], acc.at[srci.at[i]], sem_sc[s], add=True)

    def wait_scatter(i, s):
        pltpu.make_async_copy(pay.at[s], acc.at[srci.at[i]],
                              sem_sc[s]).wait()

    def compute(i, s):
        def body(b, carry):
            x0 = xd[s, b, pl.ds(0, 16)]
            x1 = xd[s, b, pl.ds(16, 16)]
            for g in range(4):
                vw = ewv[s, b, pl.ds(16 * g, 16)]
                ea = lax.bitcast_convert_type(lax.shift_left(vw, 16),
                                              jnp.float32)
                eb = lax.bitcast_convert_type(
                    jnp.bitwise_and(vw, jnp.int32(-65536)), jnp.float32)
                pay[s, b, pl.ds(32 * g, 16)] = x0 * ea
                pay[s, b, pl.ds(32 * g + 16, 16)] = x1 * eb
            return carry

        lax.fori_loop(0, _B, body, 0)

    def pair_work(i, first_pred, prefetch):
        start_in(i + 1, 1)
        wait_in(i, 0)
        if first_pred is None:
            wait_scatter(i - 2, 0)
        else:
            @pl.when(jnp.logical_not(first_pred))
            def _ws0():
                wait_scatter(i - 2, 0)
        compute(i, 0)
        start_scatter(i, 0)
        if prefetch:
            start_in(i + 2, 0)
        wait_in(i + 1, 1)
        if first_pred is None:
            wait_scatter(i - 1, 1)
        else:
            @pl.when(jnp.logical_not(first_pred))
            def _ws1():
                wait_scatter(i - 1, 1)
        compute(i + 1, 1)
        start_scatter(i + 1, 1)

    start_in(0, 0)

    def pair(p, carry):
        pair_work(2 * p, p == 0, True)
        return carry

    lax.fori_loop(0, _NB // 2 - 1, pair, 0)
    pair_work(_NB - 2, None, False)
    wait_scatter(_NB - 2, 0)
    wait_scatter(_NB - 1, 1)

    plsc.subcore_barrier()
    pltpu.sync_copy(acc.at[pl.ds(sid * _RPT, _RPT)],
                    out_hbm.at[cid, pl.ds(sid * _RPT, _RPT)])

    @pl.when(sid == 0)
    def _drain_tail():
        pltpu.sync_copy(acc.at[pl.ds(_NS * _RPT, _RTAIL)],
                        out_hbm.at[cid, pl.ds(_NS * _RPT, _RTAIL)])


def _sc_scatter(x, ew, src2d, dst2d, zeros):
    mesh = plsc.VectorSubcoreMesh(core_axis_name="c", subcore_axis_name="s")
    f = functools.partial(
        pl.kernel,
        out_type=jax.ShapeDtypeStruct((_NC, _N, 4 * _MUL), jnp.float32),
        mesh=mesh,
        compiler_params=pltpu.CompilerParams(use_tc_tiling_on_sc=False),
        scratch_types=[
            pltpu.VMEM((_NB, _B), jnp.int32),
            pltpu.VMEM((_NB, _B), jnp.int32),
            pltpu.VMEM((2, _B, _MUL), jnp.float32),
            pltpu.VMEM((2, _B, 2 * _MUL), jnp.int32),
            pltpu.VMEM((2, _B, 4 * _MUL), jnp.float32),
            pltpu.VMEM_SHARED((_N, 4 * _MUL), jnp.float32),
            pltpu.SemaphoreType.DMA,
            pltpu.SemaphoreType.DMA,
            pltpu.SemaphoreType.DMA,
            pltpu.SemaphoreType.DMA,
        ],
    )(_sc_body)
    return f(x, ew, src2d, dst2d, zeros)


_BN = 2000


def _finish_body(parts_ref, nf_ref, wskip_ref, wnl20_ref, wnl21_ref,
                 outs_ref, outv_ref):
    a = parts_ref[0] + parts_ref[1]
    s = a[:, :_MUL]
    g0 = jnp.dot(s, wnl20_ref[...],
                 preferred_element_type=jnp.float32) * _INV_SQRT_MUL
    skip = jnp.dot(nf_ref[...], wskip_ref[...],
                   preferred_element_type=jnp.float32) * _INV_SQRT_MUL
    gs = g0 + skip
    outs_ref[...] = jax.nn.silu(gs[:, :32])
    gates = jax.nn.sigmoid(gs[:, 32:48])
    row = lax.broadcasted_iota(jnp.int32, (16, 48), 0)
    col = lax.broadcasted_iota(jnp.int32, (16, 48), 1)
    acc48 = jnp.zeros((a.shape[0], 48), jnp.float32)
    for c in range(3):
        g1c = jnp.dot(a[:, _MUL * (c + 1):_MUL * (c + 2)], wnl21_ref[...],
                      preferred_element_type=jnp.float32) * _INV_SQRT_MUL
        sel = (col == 3 * row + c).astype(jnp.float32)
        acc48 = acc48 + jnp.dot(g1c * gates, sel,
                                preferred_element_type=jnp.float32)
    outv_ref[...] = acc48


def _finish(parts, nf, w_skip0, w_nl2_0, w_nl2_1):
    grid = _N // _BN
    return pl.pallas_call(
        _finish_body,
        grid=(grid,),
        in_specs=[
            pl.BlockSpec((_NC, _BN, 4 * _MUL), lambda i: (0, i, 0)),
            pl.BlockSpec((_BN, _MUL), lambda i: (i, 0)),
            pl.BlockSpec((_MUL, 48), lambda i: (0, 0)),
            pl.BlockSpec((_MUL, 48), lambda i: (0, 0)),
            pl.BlockSpec((_MUL, 16), lambda i: (0, 0)),
        ],
        out_specs=[
            pl.BlockSpec((_BN, 32), lambda i: (i, 0)),
            pl.BlockSpec((_BN, 48), lambda i: (i, 0)),
        ],
        out_shape=[
            jax.ShapeDtypeStruct((_N, 32), jnp.float32),
            jax.ShapeDtypeStruct((_N, 48), jnp.float32),
        ],
    )(parts, nf, w_skip0, w_nl2_0, w_nl2_1)


def kernel(node_feature, edge_index, edge_vec, edge_embedding,
           W_fc, W_nl0, W_skip0, W_nl2_0, W_nl2_1):
    x = _node_linear(node_feature, W_nl0)
    wfc0 = W_fc[:, :_MUL] * _INV_SQRT_EDGE
    wfc1 = W_fc[:, _MUL:2 * _MUL] * _INV_SQRT_EDGE
    wfc4 = jnp.concatenate([wfc0, wfc1, wfc1, wfc1], axis=1)
    j = jnp.arange(4 * _MUL)
    perm = (j // 32) * 32 + 16 * (j % 2) + (j % 32) // 2
    ew = _edge_prep(edge_embedding.T, edge_vec.T, wfc4[:, perm])
    ew = lax.bitcast_convert_type(ew.reshape(_E, 2 * _MUL, 2), jnp.int32)
    src2d = edge_index[0].reshape(_E // _B, _B)
    dst2d = edge_index[1].reshape(_E // _B, _B)
    zeros = jnp.zeros((_RPT, 4 * _MUL), jnp.float32)
    parts = _sc_scatter(x, ew, src2d, dst2d, zeros)
    outs, outv = _finish(parts, node_feature, W_skip0, W_nl2_0, W_nl2_1)
    return jnp.concatenate([outs, outv], axis=1)


# speedup vs baseline: 3.5855x; 3.5855x over previous
import functools

import jax
import jax.numpy as jnp
from jax import lax
from jax.experimental import pallas as pl
from jax.experimental.pallas import tpu as pltpu
from jax.experimental.pallas import tpu_sc as plsc

_N = 10000
_E = 320000
_MUL = 32
_EDGE_DIM = 12

_NC = 2
_NS = 16
_NW = _NC * _NS
_EPW = _E // _NW
_B = 40
_NB = _EPW // _B
_RPT = 624
_RTAIL = _N - _NS * _RPT

_INV_SQRT_MUL = 1.0 / (32.0 ** 0.5)
_INV_SQRT_EDGE = 1.0 / (12.0 ** 0.5)
_SQRT3 = 3.0 ** 0.5


def _node_linear_body(nf_ref, w_ref, x_ref):
    x_ref[...] = jnp.dot(nf_ref[...], w_ref[...],
                         preferred_element_type=jnp.float32) * _INV_SQRT_MUL


def _node_linear(nf, w_nl0):
    return pl.pallas_call(
        _node_linear_body,
        out_shape=jax.ShapeDtypeStruct((_N, _MUL), jnp.float32),
    )(nf, w_nl0)


_BE = 2560


def _edge_prep_body(embt_ref, vect_ref, wfc4_ref, ew_ref):
    vt = vect_ref[...]
    n2 = jnp.sum(vt * vt, axis=0, keepdims=True)
    s = _SQRT3 / (jnp.sqrt(n2) + 1e-12)
    y4 = jnp.concatenate([jnp.ones_like(s), vt * s], axis=0)
    dn = (((0,), (0,)), ((), ()))
    w128 = lax.dot_general(embt_ref[...], wfc4_ref[...], dn,
                           preferred_element_type=jnp.float32)
    row = lax.broadcasted_iota(jnp.int32, (4, 4 * _MUL), 0)
    col = lax.broadcasted_iota(jnp.int32, (4, 4 * _MUL), 1)
    sel4 = (col // _MUL == row).astype(jnp.float32)
    ybc = lax.dot_general(y4, sel4, dn,
                          preferred_element_type=jnp.float32)
    ew_ref[...] = w128 * ybc


def _edge_prep(embt, vect, wfc4):
    grid = _E // _BE
    return pl.pallas_call(
        _edge_prep_body,
        grid=(grid,),
        in_specs=[
            pl.BlockSpec((_EDGE_DIM, _BE), lambda i: (0, i)),
            pl.BlockSpec((3, _BE), lambda i: (0, i)),
            pl.BlockSpec((_EDGE_DIM, 4 * _MUL), lambda i: (0, 0)),
        ],
        out_specs=pl.BlockSpec((_BE, 4 * _MUL), lambda i: (i, 0)),
        out_shape=jax.ShapeDtypeStruct((_E, 4 * _MUL), jnp.float32),
    )(embt, vect, wfc4)


def _sc_body(x_hbm, ew_hbm, src_hbm, dst_hbm, z_hbm, out_hbm,
             srci, dsti, xd, ewv, pay, acc,
             sem_in0, sem_in1, sem_sc0, sem_sc1):
    cid = lax.axis_index("c")
    sid = lax.axis_index("s")
    wid = cid * _NS + sid
    row0 = wid * _NB

    sem_in = [sem_in0, sem_in1]
    sem_sc = [sem_sc0, sem_sc1]

    pltpu.sync_copy(src_hbm.at[pl.ds(row0, _NB)], srci)
    pltpu.sync_copy(dst_hbm.at[pl.ds(row0, _NB)], dsti)

    pltpu.sync_copy(z_hbm, acc.at[pl.ds(sid * _RPT, _RPT)])

    @pl.when(sid == 0)
    def _zero_tail():
        pltpu.sync_copy(z_hbm.at[pl.ds(0, _RTAIL)],
                        acc.at[pl.ds(_NS * _RPT, _RTAIL)])

    plsc.subcore_barrier()

    def start_in(i, s):
        pltpu.async_copy(x_hbm.at[dsti.at[i]], xd.at[s], sem_in[s])
        pltpu.async_copy(ew_hbm.at[pl.ds((row0 + i) * _B, _B)], ewv.at[s],
                         sem_in[s])

    def wait_in(i, s):
        pltpu.make_async_copy(x_hbm.at[dsti.at[i]], xd.at[s],
                              sem_in[s]).wait()
        pltpu.make_async_copy(ew_hbm.at[pl.ds((row0 + i) * _B, _B)],
                              ewv.at[s], sem_in[s]).wait()

    def start_scatter(i, s):
        pltpu.async_copy(pay.at[s], acc.at[srci.at[i]], sem_sc[s], add=True)

    def wait_scatter(i, s):
        pltpu.make_async_copy(pay.at[s], acc.at[srci.at[i]],
                              sem_sc[s]).wait()

    def compute(i, s):
        def body(b, carry):
            x0 = xd[s, b, pl.ds(0, 16)]
            x1 = xd[s, b, pl.ds(16, 16)]
            for kk in range(8):
                xk = x0 if kk % 2 == 0 else x1
                pay[s, b, pl.ds(16 * kk, 16)] = (
                    xk * ewv[s, b, pl.ds(16 * kk, 16)])
            return carry

        lax.fori_loop(0, _B, body, 0)

    def pair_work(i, first_pred, prefetch):
        start_in(i + 1, 1)
        wait_in(i, 0)
        if first_pred is None:
            wait_scatter(i - 2, 0)
        else:
            @pl.when(jnp.logical_not(first_pred))
            def _ws0():
                wait_scatter(i - 2, 0)
        compute(i, 0)
        start_scatter(i, 0)
        if prefetch:
            start_in(i + 2, 0)
        wait_in(i + 1, 1)
        if first_pred is None:
            wait_scatter(i - 1, 1)
        else:
            @pl.when(jnp.logical_not(first_pred))
            def _ws1():
                wait_scatter(i - 1, 1)
        compute(i + 1, 1)
        start_scatter(i + 1, 1)

    start_in(0, 0)

    def pair(p, carry):
        pair_work(2 * p, p == 0, True)
        return carry

    lax.fori_loop(0, _NB // 2 - 1, pair, 0)
    pair_work(_NB - 2, None, False)
    wait_scatter(_NB - 2, 0)
    wait_scatter(_NB - 1, 1)

    plsc.subcore_barrier()
    pltpu.sync_copy(acc.at[pl.ds(sid * _RPT, _RPT)],
                    out_hbm.at[cid, pl.ds(sid * _RPT, _RPT)])

    @pl.when(sid == 0)
    def _drain_tail():
        pltpu.sync_copy(acc.at[pl.ds(_NS * _RPT, _RTAIL)],
                        out_hbm.at[cid, pl.ds(_NS * _RPT, _RTAIL)])


def _sc_scatter(x, ew, src2d, dst2d, zeros):
    mesh = plsc.VectorSubcoreMesh(core_axis_name="c", subcore_axis_name="s")
    f = functools.partial(
        pl.kernel,
        out_type=jax.ShapeDtypeStruct((_NC, _N, 4 * _MUL), jnp.float32),
        mesh=mesh,
        compiler_params=pltpu.CompilerParams(use_tc_tiling_on_sc=False),
        scratch_types=[
            pltpu.VMEM((_NB, _B), jnp.int32),
            pltpu.VMEM((_NB, _B), jnp.int32),
            pltpu.VMEM((2, _B, _MUL), jnp.float32),
            pltpu.VMEM((2, _B, 4 * _MUL), jnp.float32),
            pltpu.VMEM((2, _B, 4 * _MUL), jnp.float32),
            pltpu.VMEM_SHARED((_N, 4 * _MUL), jnp.float32),
            pltpu.SemaphoreType.DMA,
            pltpu.SemaphoreType.DMA,
            pltpu.SemaphoreType.DMA,
            pltpu.SemaphoreType.DMA,
        ],
    )(_sc_body)
    return f(x, ew, src2d, dst2d, zeros)


_BN = 2000


def _finish_body(parts_ref, nf_ref, wskip_ref, wnl20_ref, wnl21_ref,
                 outs_ref, outv_ref):
    a = parts_ref[0] + parts_ref[1]
    s = a[:, :_MUL]
    g0 = jnp.dot(s, wnl20_ref[...],
                 preferred_element_type=jnp.float32) * _INV_SQRT_MUL
    skip = jnp.dot(nf_ref[...], wskip_ref[...],
                   preferred_element_type=jnp.float32) * _INV_SQRT_MUL
    gs = g0 + skip
    outs_ref[...] = jax.nn.silu(gs[:, :32])
    gates = jax.nn.sigmoid(gs[:, 32:48])
    row = lax.broadcasted_iota(jnp.int32, (16, 48), 0)
    col = lax.broadcasted_iota(jnp.int32, (16, 48), 1)
    acc48 = jnp.zeros((a.shape[0], 48), jnp.float32)
    for c in range(3):
        g1c = jnp.dot(a[:, _MUL * (c + 1):_MUL * (c + 2)], wnl21_ref[...],
                      preferred_element_type=jnp.float32) * _INV_SQRT_MUL
        sel = (col == 3 * row + c).astype(jnp.float32)
        acc48 = acc48 + jnp.dot(g1c * gates, sel,
                                preferred_element_type=jnp.float32)
    outv_ref[...] = acc48


def _finish(parts, nf, w_skip0, w_nl2_0, w_nl2_1):
    grid = _N // _BN
    return pl.pallas_call(
        _finish_body,
        grid=(grid,),
        in_specs=[
            pl.BlockSpec((_NC, _BN, 4 * _MUL), lambda i: (0, i, 0)),
            pl.BlockSpec((_BN, _MUL), lambda i: (i, 0)),
            pl.BlockSpec((_MUL, 48), lambda i: (0, 0)),
            pl.BlockSpec((_MUL, 48), lambda i: (0, 0)),
            pl.BlockSpec((_MUL, 16), lambda i: (0, 0)),
        ],
        out_specs=[
            pl.BlockSpec((_BN, 32), lambda i: (i, 0)),
            pl.BlockSpec((_BN, 48), lambda i: (i, 0)),
        ],
        out_shape=[
            jax.ShapeDtypeStruct((_N, 32), jnp.float32),
            jax.ShapeDtypeStruct((_N, 48), jnp.float32),
        ],
    )(parts, nf, w_skip0, w_nl2_0, w_nl2_1)


def kernel(node_feature, edge_index, edge_vec, edge_embedding,
           W_fc, W_nl0, W_skip0, W_nl2_0, W_nl2_1):
    x = _node_linear(node_feature, W_nl0)
    wfc0 = W_fc[:, :_MUL] * _INV_SQRT_EDGE
    wfc1 = W_fc[:, _MUL:2 * _MUL] * _INV_SQRT_EDGE
    wfc4 = jnp.concatenate([wfc0, wfc1, wfc1, wfc1], axis=1)
    ew = _edge_prep(edge_embedding.T, edge_vec.T, wfc4)
    src2d = edge_index[0].reshape(_E // _B, _B)
    dst2d = edge_index[1].reshape(_E // _B, _B)
    zeros = jnp.zeros((_RPT, 4 * _MUL), jnp.float32)
    parts = _sc_scatter(x, ew, src2d, dst2d, zeros)
    outs, outv = _finish(parts, node_feature, W_skip0, W_nl2_0, W_nl2_1)
    return jnp.concatenate([outs, outv], axis=1)
